# Initial kernel scaffold; baseline (speedup 1.0000x reference)
import jax
import jax.numpy as jnp
from jax.experimental import pallas as pl


def kernel(x, gate_w, w1, w1_up, w2):
    raise NotImplementedError("write your pallas kernel here")


# speedup vs baseline: 1.5503x; 1.5503x over previous
import functools

import jax
import jax.numpy as jnp
from jax.experimental import pallas as pl
from jax.experimental.pallas import tpu as pltpu

T = 2048
D = 1024
F = 1408
E = 8
K = 2

BT = 512
FB = 1408
NT = T // BT
NF = F // FB


def _top2_comb(x_blk, gate_w):
    logits = jax.lax.dot_general(
        x_blk, gate_w, (((1,), (1,)), ((), ())),
        preferred_element_type=jnp.float32)
    m = jnp.max(logits, axis=1, keepdims=True)
    ex = jnp.exp(logits - m)
    probs = ex / jnp.sum(ex, axis=1, keepdims=True)
    iota_e = jax.lax.broadcasted_iota(jnp.int32, probs.shape, 1)
    m1 = jnp.max(probs, axis=1, keepdims=True)
    i1 = jnp.min(jnp.where(probs == m1, iota_e, E), axis=1, keepdims=True)
    masked = jnp.where(iota_e == i1, -jnp.inf, probs)
    m2 = jnp.max(masked, axis=1, keepdims=True)
    i2 = jnp.min(jnp.where(masked == m2, iota_e, E), axis=1, keepdims=True)
    comb = (jnp.where(iota_e == i1, m1, 0.0)
            + jnp.where(iota_e == i2, m2, 0.0))
    return comb


def _moe_body(x_ref, gate_ref, w1_ref, w1u_ref, w2_ref, out_ref, comb_ref):
    e = pl.program_id(0)
    f = pl.program_id(1)
    tb = pl.program_id(2)
    rows = pl.ds(tb * BT, BT)
    x_blk = x_ref[rows, :]

    @pl.when(jnp.logical_and(e == 0, f == 0))
    def _():
        comb_ref[rows, :] = _top2_comb(x_blk, gate_ref[...])

    h = jax.lax.dot_general(x_blk, w1_ref[0], (((1,), (1,)), ((), ())),
                            preferred_element_type=jnp.float32)
    u = jax.lax.dot_general(x_blk, w1u_ref[0], (((1,), (1,)), ((), ())),
                            preferred_element_type=jnp.float32)
    g = h * (1.0 / (1.0 + jnp.exp(-h))) * u
    y = jax.lax.dot_general(g, w2_ref[0], (((1,), (1,)), ((), ())),
                            preferred_element_type=jnp.float32)
    comb_blk = comb_ref[rows, :]
    iota_e = jax.lax.broadcasted_iota(jnp.int32, comb_blk.shape, 1)
    c_e = jnp.sum(jnp.where(iota_e == e, comb_blk, 0.0), axis=1, keepdims=True)
    yw = y * c_e

    @pl.when(jnp.logical_and(e == 0, f == 0))
    def _():
        out_ref[rows, :] = yw

    @pl.when(jnp.logical_not(jnp.logical_and(e == 0, f == 0)))
    def _():
        out_ref[rows, :] += yw


@jax.jit
def kernel(x, gate_w, w1, w1_up, w2):
    return pl.pallas_call(
        _moe_body,
        grid=(E, NF, NT),
        in_specs=[
            pl.BlockSpec((T, D), lambda e, f, tb: (0, 0)),
            pl.BlockSpec((E, D), lambda e, f, tb: (0, 0)),
            pl.BlockSpec((1, FB, D), lambda e, f, tb: (e, f, 0)),
            pl.BlockSpec((1, FB, D), lambda e, f, tb: (e, f, 0)),
            pl.BlockSpec((1, D, FB), lambda e, f, tb: (e, 0, f)),
        ],
        out_specs=pl.BlockSpec((T, D), lambda e, f, tb: (0, 0)),
        out_shape=jax.ShapeDtypeStruct((T, D), jnp.float32),
        scratch_shapes=[pltpu.VMEM((T, E), jnp.float32)],
        compiler_params=pltpu.CompilerParams(
            dimension_semantics=("arbitrary", "arbitrary", "arbitrary"),
        ),
    )(x, gate_w, w1, w1_up, w2)
